# Initial kernel scaffold; baseline (speedup 1.0000x reference)
import jax
import jax.numpy as jnp
from jax.experimental import pallas as pl


def kernel(atom_output, bond_output, original_f_atoms, original_f_bonds, a2a, a2b, b2a, b2revb, a_scope, W_aa1, b_aa1, W_aa2, b_aa2, ln_aa_g, ln_aa_b, W_ab1, b_ab1, W_ab2, b_ab2, ln_ab_g, ln_ab_b, W_m1, b_m1, W_m2, b_m2):
    raise NotImplementedError("write your pallas kernel here")


# speedup vs baseline: 1.9695x; 1.9695x over previous
import functools

import jax
import jax.numpy as jnp
from jax import lax
from jax.experimental import pallas as pl
from jax.experimental.pallas import tpu as pltpu
from jax.experimental.pallas import tpu_sc as plsc

N_ATOMS = 10000
H = 128
NBR = 32
N_MOLS = 200
MOL_SIZE = 50
FFN_HID = 512
MOL_HID = 256
NUM_TASKS = 12

NW = 32
APW = 320
NPAD = NW * APW
CHUNK_ATOMS = 4
NCHUNK = APW // CHUNK_ATOMS
ROWS = CHUNK_ATOMS * NBR


def _sc_gather_sum(idx_grp, table):
    mesh = plsc.VectorSubcoreMesh(core_axis_name="c", subcore_axis_name="s")

    @functools.partial(
        pl.kernel,
        mesh=mesh,
        out_type=jax.ShapeDtypeStruct((NPAD, H), jnp.float32),
        scratch_types=[
            pltpu.VMEM((NCHUNK, ROWS), jnp.int32),
            pltpu.VMEM((ROWS, H), jnp.float32),
            pltpu.VMEM((APW, H), jnp.float32),
            pltpu.SemaphoreType.DMA,
        ],
    )
    def k(idx_hbm, table_hbm, out_hbm, idx_v, rows_v, acc_v, sem):
        c = lax.axis_index("c")
        s = lax.axis_index("s")
        wid = c * 16 + s
        pltpu.sync_copy(idx_hbm.at[wid], idx_v)

        def chunk(ci, carry):
            pltpu.async_copy(table_hbm.at[idx_v.at[ci]], rows_v, sem).wait()
            for a in range(CHUNK_ATOMS):
                base = a * NBR
                for g in range(H // 16):
                    acc = rows_v[base, pl.ds(g * 16, 16)]
                    for j in range(1, NBR):
                        acc = acc + rows_v[base + j, pl.ds(g * 16, 16)]
                    acc_v[ci * CHUNK_ATOMS + a, pl.ds(g * 16, 16)] = acc
            return carry

        lax.fori_loop(0, NCHUNK, chunk, 0)
        pltpu.sync_copy(acc_v, out_hbm.at[pl.ds(wid * APW, APW)])

    return k(idx_grp, table)


ROWS_BLK = 400
NBLK = N_ATOMS // ROWS_BLK
MOLS_BLK = ROWS_BLK // MOL_SIZE


def _tc_body(orig_ref, aggr_ref, w1a_ref, w1b_ref, b1_ref, w2_ref, b2_ref,
             g_ref, b_ref, wm1_ref, bm1_ref, wm2_ref, bm2_ref,
             out_ref, macc):
    i = pl.program_id(0)
    x = jnp.dot(orig_ref[...], w1a_ref[...], preferred_element_type=jnp.float32)
    x = x + jnp.dot(aggr_ref[...], w1b_ref[...], preferred_element_type=jnp.float32)
    h = jnp.maximum(x + b1_ref[...], 0.0)
    y = jnp.dot(h, w2_ref[...], preferred_element_type=jnp.float32) + b2_ref[...]
    mu = jnp.mean(y, axis=-1, keepdims=True)
    var = jnp.mean((y - mu) * (y - mu), axis=-1, keepdims=True)
    yn = (y - mu) * lax.rsqrt(var + 1e-5) * g_ref[...] + b_ref[...]
    pooled = jnp.sum(yn.reshape(MOLS_BLK, MOL_SIZE, H), axis=1) * (1.0 / MOL_SIZE)
    macc[pl.ds(i * MOLS_BLK, MOLS_BLK), :] = pooled

    @pl.when(i == NBLK - 1)
    def _():
        m = macc[...]
        hm = jnp.maximum(
            jnp.dot(m, wm1_ref[...], preferred_element_type=jnp.float32)
            + bm1_ref[...], 0.0)
        out = jnp.dot(hm, wm2_ref[...], preferred_element_type=jnp.float32)
        out_ref[...] = (out + bm2_ref[...]) * 0.5


def _tc_ffn(orig, aggr, w1a, w1b, b1, w2, b2, g, b, wm1, bm1, wm2, bm2):
    full = lambda shape: pl.BlockSpec(shape, lambda i: (0, 0))
    return pl.pallas_call(
        _tc_body,
        grid=(NBLK,),
        in_specs=[
            pl.BlockSpec((ROWS_BLK, H), lambda i: (i, 0)),
            pl.BlockSpec((ROWS_BLK, H), lambda i: (i, 0)),
            full((H, FFN_HID)),
            full((H, FFN_HID)),
            full((1, FFN_HID)),
            full((FFN_HID, H)),
            full((1, H)),
            full((1, H)),
            full((1, H)),
            full((H, MOL_HID)),
            full((1, MOL_HID)),
            full((MOL_HID, H)),
            full((1, H)),
        ],
        out_specs=pl.BlockSpec((N_MOLS, H), lambda i: (0, 0)),
        out_shape=jax.ShapeDtypeStruct((N_MOLS, H), jnp.float32),
        scratch_shapes=[pltpu.VMEM((N_MOLS, H), jnp.float32)],
    )(orig, aggr, w1a, w1b, b1, w2, b2, g, b, wm1, bm1, wm2, bm2)


def kernel(atom_output, bond_output, original_f_atoms, original_f_bonds,
           a2a, a2b, b2a, b2revb, a_scope,
           W_aa1, b_aa1, W_aa2, b_aa2, ln_aa_g, ln_aa_b,
           W_ab1, b_ab1, W_ab2, b_ab2, ln_ab_g, ln_ab_b,
           W_m1, b_m1, W_m2, b_m2):
    idx = jnp.zeros((NPAD, NBR), jnp.int32).at[:N_ATOMS].set(
        a2a.astype(jnp.int32))
    idx = idx.reshape(NW, NCHUNK, ROWS)
    aggr = _sc_gather_sum(idx, atom_output)

    w1a = W_aa1[:H]
    w1b = W_aa1[H:]
    wm2 = jnp.zeros((MOL_HID, H), jnp.float32).at[:, :NUM_TASKS].set(W_m2)
    bm2 = jnp.zeros((1, H), jnp.float32).at[0, :NUM_TASKS].set(b_m2)
    out = _tc_ffn(original_f_atoms, aggr, w1a, w1b,
                  b_aa1.reshape(1, -1), W_aa2, b_aa2.reshape(1, -1),
                  ln_aa_g.reshape(1, -1), ln_aa_b.reshape(1, -1),
                  W_m1, b_m1.reshape(1, -1), wm2, bm2)
    return out[:, :NUM_TASKS]


# speedup vs baseline: 2.5685x; 1.3041x over previous
import functools

import jax
import jax.numpy as jnp
from jax import lax
from jax.experimental import pallas as pl
from jax.experimental.pallas import tpu as pltpu
from jax.experimental.pallas import tpu_sc as plsc

N_ATOMS = 10000
H = 128
NBR = 32
N_MOLS = 200
MOL_SIZE = 50
FFN_HID = 512
MOL_HID = 256
NUM_TASKS = 12

NW = 32
APW = 320
NPAD = NW * APW
CHUNK_ATOMS = 4
NCHUNK = APW // CHUNK_ATOMS
ROWS = CHUNK_ATOMS * NBR


APS = 16 * APW


def _sc_gather_sum(idx_grp, own_grp, table):
    mesh = plsc.VectorSubcoreMesh(core_axis_name="c", subcore_axis_name="s")

    @functools.partial(
        pl.kernel,
        mesh=mesh,
        out_type=jax.ShapeDtypeStruct((NPAD, H), jnp.float32),
        scratch_types=[
            pltpu.VMEM((NCHUNK, ROWS), jnp.int32),
            pltpu.VMEM((NCHUNK, ROWS), jnp.int32),
            pltpu.VMEM((ROWS, H), jnp.float32),
            pltpu.VMEM((ROWS, H), jnp.float32),
            pltpu.VMEM_SHARED((APS, H), jnp.float32),
            pltpu.SemaphoreType.DMA,
            pltpu.SemaphoreType.DMA,
        ],
    )
    def k(idx_hbm, own_hbm, table_hbm, out_hbm,
          idx_v, own_v, rows_a, rows_b, acc_sh, sem_a, sem_b):
        c = lax.axis_index("c")
        s = lax.axis_index("s")
        wid = c * 16 + s
        pltpu.sync_copy(idx_hbm.at[wid], idx_v)
        pltpu.sync_copy(own_hbm.at[wid], own_v)

        zero = jnp.zeros((16,), jnp.float32)

        def zrow(r, carry):
            for g in range(H // 16):
                rows_a[r, pl.ds(g * 16, 16)] = zero
            return carry

        lax.fori_loop(0, ROWS, zrow, 0)
        pltpu.sync_copy(rows_a, acc_sh.at[pl.ds(s * APW, ROWS)])
        pltpu.sync_copy(rows_a, acc_sh.at[pl.ds(s * APW + ROWS, ROWS)])
        pltpu.sync_copy(rows_a.at[pl.ds(0, APW - 2 * ROWS)],
                        acc_sh.at[pl.ds(s * APW + 2 * ROWS, APW - 2 * ROWS)])
        plsc.subcore_barrier()

        pltpu.async_copy(table_hbm.at[idx_v.at[0]], rows_a, sem_a)

        def pair(p, carry):
            ci = p * 2
            pltpu.async_copy(table_hbm.at[idx_v.at[ci + 1]], rows_b, sem_b)
            pltpu.make_async_copy(table_hbm.at[idx_v.at[ci]], rows_a,
                                  sem_a).wait()
            pltpu.sync_copy(rows_a, acc_sh.at[own_v.at[ci]], add=True)

            @pl.when(ci + 2 < NCHUNK)
            def _():
                pltpu.async_copy(table_hbm.at[idx_v.at[ci + 2]], rows_a, sem_a)

            pltpu.make_async_copy(table_hbm.at[idx_v.at[ci + 1]], rows_b,
                                  sem_b).wait()
            pltpu.sync_copy(rows_b, acc_sh.at[own_v.at[ci + 1]], add=True)
            return carry

        lax.fori_loop(0, NCHUNK // 2, pair, 0)
        plsc.subcore_barrier()
        pltpu.sync_copy(acc_sh.at[pl.ds(s * APW, APW)],
                        out_hbm.at[pl.ds(wid * APW, APW)])

    return k(idx_grp, own_grp, table)


ROWS_BLK = 400
NBLK = N_ATOMS // ROWS_BLK
MOLS_BLK = ROWS_BLK // MOL_SIZE


def _tc_body(orig_ref, aggr_ref, w1a_ref, w1b_ref, b1_ref, w2_ref, b2_ref,
             g_ref, b_ref, wm1_ref, bm1_ref, wm2_ref, bm2_ref,
             out_ref, macc):
    i = pl.program_id(0)
    x = jnp.dot(orig_ref[...], w1a_ref[...], preferred_element_type=jnp.float32)
    x = x + jnp.dot(aggr_ref[...], w1b_ref[...], preferred_element_type=jnp.float32)
    h = jnp.maximum(x + b1_ref[...], 0.0)
    y = jnp.dot(h, w2_ref[...], preferred_element_type=jnp.float32) + b2_ref[...]
    mu = jnp.mean(y, axis=-1, keepdims=True)
    var = jnp.mean((y - mu) * (y - mu), axis=-1, keepdims=True)
    yn = (y - mu) * lax.rsqrt(var + 1e-5) * g_ref[...] + b_ref[...]
    pooled = jnp.sum(yn.reshape(MOLS_BLK, MOL_SIZE, H), axis=1) * (1.0 / MOL_SIZE)
    macc[pl.ds(i * MOLS_BLK, MOLS_BLK), :] = pooled

    @pl.when(i == NBLK - 1)
    def _():
        m = macc[...]
        hm = jnp.maximum(
            jnp.dot(m, wm1_ref[...], preferred_element_type=jnp.float32)
            + bm1_ref[...], 0.0)
        out = jnp.dot(hm, wm2_ref[...], preferred_element_type=jnp.float32)
        out_ref[...] = (out + bm2_ref[...]) * 0.5


def _tc_ffn(orig, aggr, w1a, w1b, b1, w2, b2, g, b, wm1, bm1, wm2, bm2):
    full = lambda shape: pl.BlockSpec(shape, lambda i: (0, 0))
    return pl.pallas_call(
        _tc_body,
        grid=(NBLK,),
        in_specs=[
            pl.BlockSpec((ROWS_BLK, H), lambda i: (i, 0)),
            pl.BlockSpec((ROWS_BLK, H), lambda i: (i, 0)),
            full((H, FFN_HID)),
            full((H, FFN_HID)),
            full((1, FFN_HID)),
            full((FFN_HID, H)),
            full((1, H)),
            full((1, H)),
            full((1, H)),
            full((H, MOL_HID)),
            full((1, MOL_HID)),
            full((MOL_HID, H)),
            full((1, H)),
        ],
        out_specs=pl.BlockSpec((N_MOLS, H), lambda i: (0, 0)),
        out_shape=jax.ShapeDtypeStruct((N_MOLS, H), jnp.float32),
        scratch_shapes=[pltpu.VMEM((N_MOLS, H), jnp.float32)],
    )(orig, aggr, w1a, w1b, b1, w2, b2, g, b, wm1, bm1, wm2, bm2)


def kernel(atom_output, bond_output, original_f_atoms, original_f_bonds,
           a2a, a2b, b2a, b2revb, a_scope,
           W_aa1, b_aa1, W_aa2, b_aa2, ln_aa_g, ln_aa_b,
           W_ab1, b_ab1, W_ab2, b_ab2, ln_ab_g, ln_ab_b,
           W_m1, b_m1, W_m2, b_m2):
    idx = jnp.zeros((NPAD, NBR), jnp.int32).at[:N_ATOMS].set(
        a2a.astype(jnp.int32))
    idx = idx.reshape(NW, NCHUNK, ROWS)
    own = jnp.broadcast_to(
        ((jnp.arange(NPAD, dtype=jnp.int32) % APS)[:, None]), (NPAD, NBR))
    own = own.reshape(NW, NCHUNK, ROWS)
    aggr = _sc_gather_sum(idx, own, atom_output)

    w1a = W_aa1[:H]
    w1b = W_aa1[H:]
    wm2 = jnp.zeros((MOL_HID, H), jnp.float32).at[:, :NUM_TASKS].set(W_m2)
    bm2 = jnp.zeros((1, H), jnp.float32).at[0, :NUM_TASKS].set(b_m2)
    out = _tc_ffn(original_f_atoms, aggr, w1a, w1b,
                  b_aa1.reshape(1, -1), W_aa2, b_aa2.reshape(1, -1),
                  ln_aa_g.reshape(1, -1), ln_aa_b.reshape(1, -1),
                  W_m1, b_m1.reshape(1, -1), wm2, bm2)
    return out[:, :NUM_TASKS]
